# baseline (speedup 1.0000x reference)
import jax
import jax.numpy as jnp
from jax import lax
from jax.experimental import pallas as pl
from jax.experimental.pallas import tpu as pltpu
from jax.experimental.pallas import tpu_sc as plsc

N = 100000
D = 64
B = 16384
ENTITY_AWARE_COFF = 0.001
EWC_LAMBDA = 100.0

NC = 2
NS = 16
LANES = 16
NW = NC * NS
BPW = B // NW
CH = 128
NCHUNK = BPW // CH
NG = CH // LANES

BLK_M = 2048
GRID_M = (N + BLK_M - 1) // BLK_M
N2 = GRID_M * BLK_M


def _prep_body(xt_ref, xmt_ref, xrm_ref, ewc_ref):
    i = pl.program_id(0)

    @pl.when(i == 0)
    def _():
        ewc_ref[0, 0] = 0.0

    xt = xt_ref[...]
    d = xt - xmt_ref[...]
    col = lax.broadcasted_iota(jnp.int32, (D, BLK_M), 1)
    d = jnp.where(col < N - i * BLK_M, d, 0.0)
    ewc_ref[0, 0] += (EWC_LAMBDA / 2.0) * 1e-05 * jnp.sum(d * d)
    t = jnp.transpose(xt)
    xrm_ref[:, :D] = t[: BLK_M // 2]
    xrm_ref[:, D:] = t[BLK_M // 2:]


def _prep(xt, xmt):
    return pl.pallas_call(
        _prep_body,
        grid=(GRID_M,),
        in_specs=[
            pl.BlockSpec((D, BLK_M), lambda i: (0, i)),
            pl.BlockSpec((D, BLK_M), lambda i: (0, i)),
        ],
        out_specs=(
            pl.BlockSpec((BLK_M // 2, 2 * D), lambda i: (i, 0)),
            pl.BlockSpec(memory_space=pltpu.SMEM),
        ),
        out_shape=(
            jax.ShapeDtypeStruct((N2 // 2, 2 * D), jnp.float32),
            jax.ShapeDtypeStruct((1, 1), jnp.float32),
        ),
    )(xt, xmt)


def _sc_body(x_hbm, i0, i1, i2, i3, i4, i5, i6, cf_hbm, it_hbm, us_hbm,
             idx_v,
             b0_u, b0_p, b0_n, b0_e1, b0_e2, b0_e3, b0_e4,
             b1_u, b1_p, b1_n, b1_e1, b1_e2, b1_e3, b1_e4,
             cf_b, it_b, us_b, *sems):
    wid = lax.axis_index("s") * NC + lax.axis_index("c")
    base = wid * BPW
    banks = ((b0_u, b0_p, b0_n, b0_e1, b0_e2, b0_e3, b0_e4),
             (b1_u, b1_p, b1_n, b1_e1, b1_e2, b1_e3, b1_e4))

    for j, ih in enumerate((i0, i1, i2, i3, i4, i5, i6)):
        pltpu.sync_copy(ih.at[wid], idx_v.at[j])

    def zero(g, _):
        z = jnp.zeros((LANES,), jnp.float32)
        o = pl.ds(g * LANES, LANES)
        cf_b[o] = z
        it_b[o] = z
        us_b[o] = z
        return 0

    lax.fori_loop(0, BPW // LANES, zero, 0)

    def fire(c):
        b = banks[c % 2]
        s = sems[(c % 2) * 7:(c % 2) * 7 + 7]
        return [pltpu.async_copy(x_hbm.at[idx_v.at[j, c]], b[j], s[j])
                for j in range(7)]

    cps = fire(0)
    for c in range(NCHUNK):
        for cp in cps:
            cp.wait()
        cps = fire(c + 1) if c + 1 < NCHUNK else []
        u_r, p_r, n_r, e1_r, e2_r, e3_r, e4_r = banks[c % 2]

        def pair_step(p, _):
            cf = jnp.zeros((LANES,), jnp.float32)
            it = jnp.zeros((LANES,), jnp.float32)
            us = jnp.zeros((LANES,), jnp.float32)
            for d in range(D // LANES):
                ds_ = pl.ds(d * LANES, LANES)
                xu = u_r[p, ds_]
                xi = p_r[p, ds_]
                cf += xu * (xi - n_r[p, ds_])
                dip = xi - e1_r[p, ds_]
                din = xi - e2_r[p, ds_]
                it += dip * dip - din * din
                dup = xu - e3_r[p, ds_]
                dun = xu - e4_r[p, ds_]
                us += dup * dup - dun * dun
            rowv = jnp.full((LANES,), c * CH + p, jnp.int32)
            plsc.addupdate_scatter(cf_b, [rowv], cf)
            plsc.addupdate_scatter(it_b, [rowv], it)
            plsc.addupdate_scatter(us_b, [rowv], us)
            return 0

        lax.fori_loop(0, CH, pair_step, 0, unroll=4)

    pltpu.sync_copy(cf_b, cf_hbm.at[pl.ds(base, BPW)])
    pltpu.sync_copy(it_b, it_hbm.at[pl.ds(base, BPW)])
    pltpu.sync_copy(us_b, us_hbm.at[pl.ds(base, BPW)])


def _sc_partials(x_rm, idx_cols):
    mesh = plsc.VectorSubcoreMesh(core_axis_name="c", subcore_axis_name="s")
    f32 = jnp.float32
    out = jax.ShapeDtypeStruct((B,), f32)
    rows = pltpu.VMEM((CH, D), f32)
    outb = pltpu.VMEM((BPW,), f32)
    return pl.kernel(
        _sc_body,
        out_type=(out, out, out),
        mesh=mesh,
        compiler_params=pltpu.CompilerParams(use_tc_tiling_on_sc=False,
                                             needs_layout_passes=False),
        scratch_types=(
            [pltpu.VMEM((7, NCHUNK, CH), jnp.int32)]
            + [rows] * 14
            + [outb] * 3
            + [pltpu.SemaphoreType.DMA] * 14
        ),
    )(x_rm, *idx_cols)


def _log_sigmoid(z):
    return jnp.minimum(z, 0.0) - jnp.log1p(jnp.exp(-jnp.abs(z)))


def _final_body(cf_ref, it_ref, us_ref, mi_ref, mu_ref, ewc_ref, out_ref):
    cf_loss = -jnp.sum(_log_sigmoid(cf_ref[...]))
    reg_loss = (-jnp.sum(_log_sigmoid(it_ref[...] * mi_ref[...]))
                - jnp.sum(_log_sigmoid(us_ref[...] * mu_ref[...])))
    out_ref[0, 0] = cf_loss + ENTITY_AWARE_COFF * reg_loss + ewc_ref[0, 0]


def _finalize(cf, it, us, mi, mu, ewc):
    vec = pl.BlockSpec((B,), lambda: (0,))
    return pl.pallas_call(
        _final_body,
        in_specs=[vec, vec, vec, vec, vec,
                  pl.BlockSpec(memory_space=pltpu.SMEM)],
        out_specs=pl.BlockSpec(memory_space=pltpu.SMEM),
        out_shape=jax.ShapeDtypeStruct((1, 1), jnp.float32),
    )(cf, it, us, mi, mu, ewc)


@jax.jit
def kernel(x, pos_neg_pair_t, x_estimated_mean):
    idx_cols = []
    for j in (0, 1, 2, 3, 4, 6, 7):
        v = pos_neg_pair_t[:, j]
        v = ((v >> 11) << 11) | ((v & 1023) << 1) | ((v >> 10) & 1)
        idx_cols.append(v.reshape(NW, NCHUNK, CH))
    mi = pos_neg_pair_t[:, 5].astype(jnp.float32)
    mu = pos_neg_pair_t[:, 8].astype(jnp.float32)

    x_rm, ewc = _prep(x.T, x_estimated_mean.T)
    cf, it, us = _sc_partials(x_rm.reshape(N2, D), idx_cols)
    loss = _finalize(cf, it, us, mi, mu, ewc)
    return loss[0, 0]


# speedup vs baseline: 1.1417x; 1.1417x over previous
import jax
import jax.numpy as jnp
from jax import lax
from jax.experimental import pallas as pl
from jax.experimental.pallas import tpu as pltpu
from jax.experimental.pallas import tpu_sc as plsc

N = 100000
D = 64
B = 16384
ENTITY_AWARE_COFF = 0.001
EWC_LAMBDA = 100.0

NC = 2
NS = 16
LANES = 16
NW = NC * NS
BPW = B // NW
CH = 128
NCHUNK = BPW // CH
NG = CH // LANES

BLK_M = 2048
GRID_M = (N + BLK_M - 1) // BLK_M
N2 = GRID_M * BLK_M


def _prep_body(xt_ref, xmt_ref, xrm_ref, ewc_ref):
    i = pl.program_id(0)

    @pl.when(i == 0)
    def _():
        ewc_ref[0, 0] = 0.0

    xt = xt_ref[...]
    d = xt - xmt_ref[...]
    col = lax.broadcasted_iota(jnp.int32, (D, BLK_M), 1)
    d = jnp.where(col < N - i * BLK_M, d, 0.0)
    ewc_ref[0, 0] += (EWC_LAMBDA / 2.0) * 1e-05 * jnp.sum(d * d)
    t = jnp.transpose(xt)
    xrm_ref[:, :D] = t[: BLK_M // 2]
    xrm_ref[:, D:] = t[BLK_M // 2:]


def _prep(xt, xmt):
    return pl.pallas_call(
        _prep_body,
        grid=(GRID_M,),
        in_specs=[
            pl.BlockSpec((D, BLK_M), lambda i: (0, i)),
            pl.BlockSpec((D, BLK_M), lambda i: (0, i)),
        ],
        out_specs=(
            pl.BlockSpec((BLK_M // 2, 2 * D), lambda i: (i, 0)),
            pl.BlockSpec(memory_space=pltpu.SMEM),
        ),
        out_shape=(
            jax.ShapeDtypeStruct((N2 // 2, 2 * D), jnp.float32),
            jax.ShapeDtypeStruct((1, 1), jnp.float32),
        ),
    )(xt, xmt)


def _sc_body(x_hbm, i0, i1, i2, i3, i4, i5, i6, cf_hbm, it_hbm, us_hbm,
             idx_v,
             b0_u, b0_p, b0_n, b0_e1, b0_e2, b0_e3, b0_e4,
             b1_u, b1_p, b1_n, b1_e1, b1_e2, b1_e3, b1_e4,
             st_cf, st_it, st_us, cf_b, it_b, us_b, *sems):
    wid = lax.axis_index("s") * NC + lax.axis_index("c")
    base = wid * BPW
    lane = lax.iota(jnp.int32, LANES)
    banks = ((b0_u, b0_p, b0_n, b0_e1, b0_e2, b0_e3, b0_e4),
             (b1_u, b1_p, b1_n, b1_e1, b1_e2, b1_e3, b1_e4))

    for j, ih in enumerate((i0, i1, i2, i3, i4, i5, i6)):
        pltpu.sync_copy(ih.at[wid], idx_v.at[j])

    def fire(c):
        b = banks[c % 2]
        s = sems[(c % 2) * 7:(c % 2) * 7 + 7]
        return [pltpu.async_copy(x_hbm.at[idx_v.at[j, c]], b[j], s[j])
                for j in range(7)]

    cps = fire(0)
    for c in range(NCHUNK):
        for cp in cps:
            cp.wait()
        cps = fire(c + 1) if c + 1 < NCHUNK else []
        u_r, p_r, n_r, e1_r, e2_r, e3_r, e4_r = banks[c % 2]

        def pair_step(p, _):
            cf = jnp.zeros((LANES,), jnp.float32)
            it = jnp.zeros((LANES,), jnp.float32)
            us = jnp.zeros((LANES,), jnp.float32)
            for d in range(D // LANES):
                ds_ = pl.ds(d * LANES, LANES)
                xu = u_r[p, ds_]
                xi = p_r[p, ds_]
                cf += xu * (xi - n_r[p, ds_])
                dip = xi - e1_r[p, ds_]
                din = xi - e2_r[p, ds_]
                it += dip * dip - din * din
                dup = xu - e3_r[p, ds_]
                dun = xu - e4_r[p, ds_]
                us += dup * dup - dun * dun
            jj = p & (LANES - 1)
            colv = jnp.full((LANES,), jj, jnp.int32)
            plsc.store_scatter(st_cf, [lane, colv], cf)
            plsc.store_scatter(st_it, [lane, colv], it)
            plsc.store_scatter(st_us, [lane, colv], us)

            @pl.when(jj == LANES - 1)
            def _():
                o = pl.ds(c * CH + p - (LANES - 1), LANES)
                for st, out_b in ((st_cf, cf_b), (st_it, it_b), (st_us, us_b)):
                    tot = st[0, :]
                    for l in range(1, LANES):
                        tot = tot + st[l, :]
                    out_b[o] = tot

            return 0

        lax.fori_loop(0, CH, pair_step, 0, unroll=LANES)

    pltpu.sync_copy(cf_b, cf_hbm.at[pl.ds(base, BPW)])
    pltpu.sync_copy(it_b, it_hbm.at[pl.ds(base, BPW)])
    pltpu.sync_copy(us_b, us_hbm.at[pl.ds(base, BPW)])


def _sc_partials(x_rm, idx_cols):
    mesh = plsc.VectorSubcoreMesh(core_axis_name="c", subcore_axis_name="s")
    f32 = jnp.float32
    out = jax.ShapeDtypeStruct((B,), f32)
    rows = pltpu.VMEM((CH, D), f32)
    stage = pltpu.VMEM((LANES, LANES), f32)
    outb = pltpu.VMEM((BPW,), f32)
    return pl.kernel(
        _sc_body,
        out_type=(out, out, out),
        mesh=mesh,
        compiler_params=pltpu.CompilerParams(use_tc_tiling_on_sc=False,
                                             needs_layout_passes=False),
        scratch_types=(
            [pltpu.VMEM((7, NCHUNK, CH), jnp.int32)]
            + [rows] * 14
            + [stage] * 3
            + [outb] * 3
            + [pltpu.SemaphoreType.DMA] * 14
        ),
    )(x_rm, *idx_cols)


def _log_sigmoid(z):
    return jnp.minimum(z, 0.0) - jnp.log1p(jnp.exp(-jnp.abs(z)))


def _final_body(cf_ref, it_ref, us_ref, mi_ref, mu_ref, ewc_ref, out_ref):
    cf_loss = -jnp.sum(_log_sigmoid(cf_ref[...]))
    reg_loss = (-jnp.sum(_log_sigmoid(it_ref[...] * mi_ref[...]))
                - jnp.sum(_log_sigmoid(us_ref[...] * mu_ref[...])))
    out_ref[0, 0] = cf_loss + ENTITY_AWARE_COFF * reg_loss + ewc_ref[0, 0]


def _finalize(cf, it, us, mi, mu, ewc):
    vec = pl.BlockSpec((B,), lambda: (0,))
    return pl.pallas_call(
        _final_body,
        in_specs=[vec, vec, vec, vec, vec,
                  pl.BlockSpec(memory_space=pltpu.SMEM)],
        out_specs=pl.BlockSpec(memory_space=pltpu.SMEM),
        out_shape=jax.ShapeDtypeStruct((1, 1), jnp.float32),
    )(cf, it, us, mi, mu, ewc)


@jax.jit
def kernel(x, pos_neg_pair_t, x_estimated_mean):
    idx_cols = []
    for j in (0, 1, 2, 3, 4, 6, 7):
        v = pos_neg_pair_t[:, j]
        v = ((v >> 11) << 11) | ((v & 1023) << 1) | ((v >> 10) & 1)
        idx_cols.append(v.reshape(NW, NCHUNK, CH))
    mi = pos_neg_pair_t[:, 5].astype(jnp.float32)
    mu = pos_neg_pair_t[:, 8].astype(jnp.float32)

    x_rm, ewc = _prep(x.T, x_estimated_mean.T)
    cf, it, us = _sc_partials(x_rm.reshape(N2, D), idx_cols)
    loss = _finalize(cf, it, us, mi, mu, ewc)
    return loss[0, 0]


# speedup vs baseline: 1.3190x; 1.1554x over previous
import jax
import jax.numpy as jnp
from jax import lax
from jax.experimental import pallas as pl
from jax.experimental.pallas import tpu as pltpu
from jax.experimental.pallas import tpu_sc as plsc

N = 100000
D = 64
B = 16384
ENTITY_AWARE_COFF = 0.001
EWC_LAMBDA = 100.0

NC = 2
NS = 16
LANES = 16
NW = NC * NS
BPW = B // NW
CH = 128
NCHUNK = BPW // CH
NG = CH // LANES

BLK_M = 4096
GRID_M = (N + BLK_M - 1) // BLK_M
N2 = GRID_M * BLK_M


def _prep_body(xt_ref, xmt_ref, xrm_ref, ewc_ref):
    i = pl.program_id(0)

    @pl.when(i == 0)
    def _():
        ewc_ref[0, 0] = 0.0

    xt = xt_ref[...]
    d = xt - xmt_ref[...]
    col = lax.broadcasted_iota(jnp.int32, (D, BLK_M), 1)
    d = jnp.where(col < N - i * BLK_M, d, 0.0)
    ewc_ref[0, 0] += (EWC_LAMBDA / 2.0) * 1e-05 * jnp.sum(d * d)
    t = jnp.transpose(xt)
    xrm_ref[:, :D] = t[: BLK_M // 2]
    xrm_ref[:, D:] = t[BLK_M // 2:]


def _prep(xt, xmt):
    return pl.pallas_call(
        _prep_body,
        grid=(GRID_M,),
        in_specs=[
            pl.BlockSpec((D, BLK_M), lambda i: (0, i)),
            pl.BlockSpec((D, BLK_M), lambda i: (0, i)),
        ],
        out_specs=(
            pl.BlockSpec((BLK_M // 2, 2 * D), lambda i: (i, 0)),
            pl.BlockSpec(memory_space=pltpu.SMEM),
        ),
        out_shape=(
            jax.ShapeDtypeStruct((N2 // 2, 2 * D), jnp.float32),
            jax.ShapeDtypeStruct((1, 1), jnp.float32),
        ),
    )(xt, xmt)


def _sc_body(x_hbm, i0, i1, i2, i3, i4, i5, i6, cf_hbm, it_hbm, us_hbm,
             idx_v,
             b0_u, b0_p, b0_n, b0_e1, b0_e2, b0_e3, b0_e4,
             b1_u, b1_p, b1_n, b1_e1, b1_e2, b1_e3, b1_e4,
             st_cf, st_it, st_us, cf_b, it_b, us_b, *sems):
    wid = lax.axis_index("s") * NC + lax.axis_index("c")
    base = wid * BPW
    lane = lax.iota(jnp.int32, LANES)
    banks = ((b0_u, b0_p, b0_n, b0_e1, b0_e2, b0_e3, b0_e4),
             (b1_u, b1_p, b1_n, b1_e1, b1_e2, b1_e3, b1_e4))

    for j, ih in enumerate((i0, i1, i2, i3, i4, i5, i6)):
        pltpu.sync_copy(ih.at[wid], idx_v.at[j])

    def fire(c):
        b = banks[c % 2]
        s = sems[(c % 2) * 7:(c % 2) * 7 + 7]
        return [pltpu.async_copy(x_hbm.at[idx_v.at[j, c]], b[j], s[j])
                for j in range(7)]

    cps = fire(0)
    for c in range(NCHUNK):
        for cp in cps:
            cp.wait()
        cps = fire(c + 1) if c + 1 < NCHUNK else []
        u_r, p_r, n_r, e1_r, e2_r, e3_r, e4_r = banks[c % 2]

        def pair_step(p, _):
            cf = jnp.zeros((LANES,), jnp.float32)
            it = jnp.zeros((LANES,), jnp.float32)
            us = jnp.zeros((LANES,), jnp.float32)
            for d in range(D // LANES):
                ds_ = pl.ds(d * LANES, LANES)
                xu = u_r[p, ds_]
                xi = p_r[p, ds_]
                cf += xu * (xi - n_r[p, ds_])
                dip = xi - e1_r[p, ds_]
                din = xi - e2_r[p, ds_]
                it += dip * dip - din * din
                dup = xu - e3_r[p, ds_]
                dun = xu - e4_r[p, ds_]
                us += dup * dup - dun * dun
            jj = p & (LANES - 1)
            colv = jnp.full((LANES,), jj, jnp.int32)
            plsc.store_scatter(st_cf, [lane, colv], cf)
            plsc.store_scatter(st_it, [lane, colv], it)
            plsc.store_scatter(st_us, [lane, colv], us)

            @pl.when(jj == LANES - 1)
            def _():
                o = pl.ds(c * CH + p - (LANES - 1), LANES)
                for st, out_b in ((st_cf, cf_b), (st_it, it_b), (st_us, us_b)):
                    rows_ = [st[l, :] for l in range(LANES)]
                    while len(rows_) > 1:
                        rows_ = [a + b for a, b in zip(rows_[::2], rows_[1::2])]
                    out_b[o] = rows_[0]

            return 0

        lax.fori_loop(0, CH, pair_step, 0, unroll=2)

    pltpu.sync_copy(cf_b, cf_hbm.at[pl.ds(base, BPW)])
    pltpu.sync_copy(it_b, it_hbm.at[pl.ds(base, BPW)])
    pltpu.sync_copy(us_b, us_hbm.at[pl.ds(base, BPW)])


def _sc_partials(x_rm, idx_cols):
    mesh = plsc.VectorSubcoreMesh(core_axis_name="c", subcore_axis_name="s")
    f32 = jnp.float32
    out = jax.ShapeDtypeStruct((B,), f32)
    rows = pltpu.VMEM((CH, D), f32)
    stage = pltpu.VMEM((LANES, LANES), f32)
    outb = pltpu.VMEM((BPW,), f32)
    return pl.kernel(
        _sc_body,
        out_type=(out, out, out),
        mesh=mesh,
        compiler_params=pltpu.CompilerParams(use_tc_tiling_on_sc=False,
                                             needs_layout_passes=False),
        scratch_types=(
            [pltpu.VMEM((7, NCHUNK, CH), jnp.int32)]
            + [rows] * 14
            + [stage] * 3
            + [outb] * 3
            + [pltpu.SemaphoreType.DMA] * 14
        ),
    )(x_rm, *idx_cols)


def _log_sigmoid(z):
    return jnp.minimum(z, 0.0) - jnp.log1p(jnp.exp(-jnp.abs(z)))


def _final_body(cf_ref, it_ref, us_ref, mi_ref, mu_ref, ewc_ref, out_ref):
    cf_loss = -jnp.sum(_log_sigmoid(cf_ref[...]))
    reg_loss = (-jnp.sum(_log_sigmoid(it_ref[...] * mi_ref[...]))
                - jnp.sum(_log_sigmoid(us_ref[...] * mu_ref[...])))
    out_ref[0, 0] = cf_loss + ENTITY_AWARE_COFF * reg_loss + ewc_ref[0, 0]


def _finalize(cf, it, us, mi, mu, ewc):
    vec = pl.BlockSpec((B,), lambda: (0,))
    return pl.pallas_call(
        _final_body,
        in_specs=[vec, vec, vec, vec, vec,
                  pl.BlockSpec(memory_space=pltpu.SMEM)],
        out_specs=pl.BlockSpec(memory_space=pltpu.SMEM),
        out_shape=jax.ShapeDtypeStruct((1, 1), jnp.float32),
    )(cf, it, us, mi, mu, ewc)


@jax.jit
def kernel(x, pos_neg_pair_t, x_estimated_mean):
    sh = BLK_M.bit_length() - 1
    half = BLK_M // 2
    idx_cols = []
    for j in (0, 1, 2, 3, 4, 6, 7):
        v = pos_neg_pair_t[:, j]
        v = ((v >> sh) << sh) | ((v & (half - 1)) << 1) | ((v >> (sh - 1)) & 1)
        idx_cols.append(v.reshape(NW, NCHUNK, CH))
    mi = pos_neg_pair_t[:, 5].astype(jnp.float32)
    mu = pos_neg_pair_t[:, 8].astype(jnp.float32)

    x_rm, ewc = _prep(x.T, x_estimated_mean.T)
    cf, it, us = _sc_partials(x_rm.reshape(N2, D), idx_cols)
    loss = _finalize(cf, it, us, mi, mu, ewc)
    return loss[0, 0]


# speedup vs baseline: 1.4230x; 1.0788x over previous
import jax
import jax.numpy as jnp
from jax import lax
from jax.experimental import pallas as pl
from jax.experimental.pallas import tpu as pltpu
from jax.experimental.pallas import tpu_sc as plsc

N = 100000
D = 64
B = 16384
ENTITY_AWARE_COFF = 0.001
EWC_LAMBDA = 100.0

NC = 2
NS = 16
LANES = 16
NW = NC * NS
BPW = B // NW
CH = 128
NCHUNK = BPW // CH
NG = CH // LANES

BLK_M = 8192
GRID_M = (N + BLK_M - 1) // BLK_M
N2 = GRID_M * BLK_M


def _prep_body(xt_ref, xmt_ref, xrm_ref, ewc_ref):
    i = pl.program_id(0)

    @pl.when(i == 0)
    def _():
        ewc_ref[0, 0] = 0.0

    xt = xt_ref[...]
    d = xt - xmt_ref[...]
    col = lax.broadcasted_iota(jnp.int32, (D, BLK_M), 1)
    d = jnp.where(col < N - i * BLK_M, d, 0.0)
    ewc_ref[0, 0] += (EWC_LAMBDA / 2.0) * 1e-05 * jnp.sum(d * d)
    t = jnp.transpose(xt)
    xrm_ref[:, :D] = t[: BLK_M // 2]
    xrm_ref[:, D:] = t[BLK_M // 2:]


def _prep(xt, xmt):
    return pl.pallas_call(
        _prep_body,
        grid=(GRID_M,),
        in_specs=[
            pl.BlockSpec((D, BLK_M), lambda i: (0, i)),
            pl.BlockSpec((D, BLK_M), lambda i: (0, i)),
        ],
        out_specs=(
            pl.BlockSpec((BLK_M // 2, 2 * D), lambda i: (i, 0)),
            pl.BlockSpec(memory_space=pltpu.SMEM),
        ),
        out_shape=(
            jax.ShapeDtypeStruct((N2 // 2, 2 * D), jnp.float32),
            jax.ShapeDtypeStruct((1, 1), jnp.float32),
        ),
    )(xt, xmt)


def _sc_body(x_hbm, i0, i1, i2, i3, i4, i5, i6, cf_hbm, it_hbm, us_hbm,
             idx_v,
             b0_u, b0_p, b0_n, b0_e1, b0_e2, b0_e3, b0_e4,
             b1_u, b1_p, b1_n, b1_e1, b1_e2, b1_e3, b1_e4,
             st_cf, st_it, st_us, cf_b, it_b, us_b, *sems):
    wid = lax.axis_index("s") * NC + lax.axis_index("c")
    base = wid * BPW
    lane = lax.iota(jnp.int32, LANES)
    banks = ((b0_u, b0_p, b0_n, b0_e1, b0_e2, b0_e3, b0_e4),
             (b1_u, b1_p, b1_n, b1_e1, b1_e2, b1_e3, b1_e4))

    for j, ih in enumerate((i0, i1, i2, i3, i4, i5, i6)):
        pltpu.sync_copy(ih.at[wid], idx_v.at[j])

    def fire(c):
        b = banks[c % 2]
        s = sems[(c % 2) * 7:(c % 2) * 7 + 7]
        return [pltpu.async_copy(x_hbm.at[idx_v.at[j, c]], b[j], s[j])
                for j in range(7)]

    cps = fire(0)
    for c in range(NCHUNK):
        for cp in cps:
            cp.wait()
        cps = fire(c + 1) if c + 1 < NCHUNK else []
        u_r, p_r, n_r, e1_r, e2_r, e3_r, e4_r = banks[c % 2]

        def pair_step(p, _):
            cf = jnp.zeros((LANES,), jnp.float32)
            it = jnp.zeros((LANES,), jnp.float32)
            us = jnp.zeros((LANES,), jnp.float32)
            for d in range(D // LANES):
                ds_ = pl.ds(d * LANES, LANES)
                xu = u_r[p, ds_]
                xi = p_r[p, ds_]
                cf += xu * (xi - n_r[p, ds_])
                dip = xi - e1_r[p, ds_]
                din = xi - e2_r[p, ds_]
                it += dip * dip - din * din
                dup = xu - e3_r[p, ds_]
                dun = xu - e4_r[p, ds_]
                us += dup * dup - dun * dun
            jj = p & (LANES - 1)
            colv = jnp.full((LANES,), jj, jnp.int32)
            plsc.store_scatter(st_cf, [lane, colv], cf)
            plsc.store_scatter(st_it, [lane, colv], it)
            plsc.store_scatter(st_us, [lane, colv], us)

            @pl.when(jj == LANES - 1)
            def _():
                o = pl.ds(c * CH + p - (LANES - 1), LANES)
                for st, out_b in ((st_cf, cf_b), (st_it, it_b), (st_us, us_b)):
                    rows_ = [st[l, :] for l in range(LANES)]
                    while len(rows_) > 1:
                        rows_ = [a + b for a, b in zip(rows_[::2], rows_[1::2])]
                    out_b[o] = rows_[0]

            return 0

        lax.fori_loop(0, CH, pair_step, 0, unroll=2)

    pltpu.sync_copy(cf_b, cf_hbm.at[pl.ds(base, BPW)])
    pltpu.sync_copy(it_b, it_hbm.at[pl.ds(base, BPW)])
    pltpu.sync_copy(us_b, us_hbm.at[pl.ds(base, BPW)])


def _sc_partials(x_rm, idx_cols):
    mesh = plsc.VectorSubcoreMesh(core_axis_name="c", subcore_axis_name="s")
    f32 = jnp.float32
    out = jax.ShapeDtypeStruct((B,), f32)
    rows = pltpu.VMEM((CH, D), f32)
    stage = pltpu.VMEM((LANES, LANES), f32)
    outb = pltpu.VMEM((BPW,), f32)
    return pl.kernel(
        _sc_body,
        out_type=(out, out, out),
        mesh=mesh,
        compiler_params=pltpu.CompilerParams(use_tc_tiling_on_sc=False,
                                             needs_layout_passes=False),
        scratch_types=(
            [pltpu.VMEM((7, NCHUNK, CH), jnp.int32)]
            + [rows] * 14
            + [stage] * 3
            + [outb] * 3
            + [pltpu.SemaphoreType.DMA] * 14
        ),
    )(x_rm, *idx_cols)


def _log_sigmoid(z):
    return jnp.minimum(z, 0.0) - jnp.log1p(jnp.exp(-jnp.abs(z)))


def _final_body(cf_ref, it_ref, us_ref, mi_ref, mu_ref, ewc_ref, out_ref):
    cf_loss = -jnp.sum(_log_sigmoid(cf_ref[...]))
    reg_loss = (-jnp.sum(_log_sigmoid(it_ref[...] * mi_ref[...]))
                - jnp.sum(_log_sigmoid(us_ref[...] * mu_ref[...])))
    out_ref[0, 0] = cf_loss + ENTITY_AWARE_COFF * reg_loss + ewc_ref[0, 0]


def _finalize(cf, it, us, mi, mu, ewc):
    vec = pl.BlockSpec((B,), lambda: (0,))
    return pl.pallas_call(
        _final_body,
        in_specs=[vec, vec, vec, vec, vec,
                  pl.BlockSpec(memory_space=pltpu.SMEM)],
        out_specs=pl.BlockSpec(memory_space=pltpu.SMEM),
        out_shape=jax.ShapeDtypeStruct((1, 1), jnp.float32),
    )(cf, it, us, mi, mu, ewc)


@jax.jit
def kernel(x, pos_neg_pair_t, x_estimated_mean):
    sh = BLK_M.bit_length() - 1
    half = BLK_M // 2
    idx_cols = []
    for j in (0, 1, 2, 3, 4, 6, 7):
        v = pos_neg_pair_t[:, j]
        v = ((v >> sh) << sh) | ((v & (half - 1)) << 1) | ((v >> (sh - 1)) & 1)
        idx_cols.append(v.reshape(NW, NCHUNK, CH))
    mi = pos_neg_pair_t[:, 5].astype(jnp.float32)
    mu = pos_neg_pair_t[:, 8].astype(jnp.float32)

    x_rm, ewc = _prep(x.T, x_estimated_mean.T)
    cf, it, us = _sc_partials(x_rm.reshape(N2, D), idx_cols)
    loss = _finalize(cf, it, us, mi, mu, ewc)
    return loss[0, 0]


# speedup vs baseline: 1.4285x; 1.0039x over previous
import jax
import jax.numpy as jnp
from jax import lax
from jax.experimental import pallas as pl
from jax.experimental.pallas import tpu as pltpu
from jax.experimental.pallas import tpu_sc as plsc

N = 100000
D = 64
B = 16384
ENTITY_AWARE_COFF = 0.001
EWC_LAMBDA = 100.0

NC = 2
NS = 16
LANES = 16
NW = NC * NS
BPW = B // NW
CH = 128
NCHUNK = BPW // CH
NG = CH // LANES

BLK_M = 16384
GRID_M = (N + BLK_M - 1) // BLK_M
N2 = GRID_M * BLK_M


def _prep_body(xt_ref, xmt_ref, xrm_ref, ewc_ref):
    i = pl.program_id(0)

    @pl.when(i == 0)
    def _():
        ewc_ref[0, 0] = 0.0

    xt = xt_ref[...]
    d = xt - xmt_ref[...]
    col = lax.broadcasted_iota(jnp.int32, (D, BLK_M), 1)
    d = jnp.where(col < N - i * BLK_M, d, 0.0)
    ewc_ref[0, 0] += (EWC_LAMBDA / 2.0) * 1e-05 * jnp.sum(d * d)
    t = jnp.transpose(xt)
    xrm_ref[:, :D] = t[: BLK_M // 2]
    xrm_ref[:, D:] = t[BLK_M // 2:]


def _prep(xt, xmt):
    return pl.pallas_call(
        _prep_body,
        grid=(GRID_M,),
        in_specs=[
            pl.BlockSpec((D, BLK_M), lambda i: (0, i)),
            pl.BlockSpec((D, BLK_M), lambda i: (0, i)),
        ],
        out_specs=(
            pl.BlockSpec((BLK_M // 2, 2 * D), lambda i: (i, 0)),
            pl.BlockSpec(memory_space=pltpu.SMEM),
        ),
        out_shape=(
            jax.ShapeDtypeStruct((N2 // 2, 2 * D), jnp.float32),
            jax.ShapeDtypeStruct((1, 1), jnp.float32),
        ),
    )(xt, xmt)


def _sc_body(x_hbm, i0, i1, i2, i3, i4, i5, i6, cf_hbm, it_hbm, us_hbm,
             idx_v,
             b0_u, b0_p, b0_n, b0_e1, b0_e2, b0_e3, b0_e4,
             b1_u, b1_p, b1_n, b1_e1, b1_e2, b1_e3, b1_e4,
             st_cf, st_it, st_us, cf_b, it_b, us_b, *sems):
    wid = lax.axis_index("s") * NC + lax.axis_index("c")
    base = wid * BPW
    lane = lax.iota(jnp.int32, LANES)
    banks = ((b0_u, b0_p, b0_n, b0_e1, b0_e2, b0_e3, b0_e4),
             (b1_u, b1_p, b1_n, b1_e1, b1_e2, b1_e3, b1_e4))

    for j, ih in enumerate((i0, i1, i2, i3, i4, i5, i6)):
        pltpu.sync_copy(ih.at[wid], idx_v.at[j])

    def fire(c):
        b = banks[c % 2]
        s = sems[(c % 2) * 7:(c % 2) * 7 + 7]
        return [pltpu.async_copy(x_hbm.at[idx_v.at[j, c]], b[j], s[j])
                for j in range(7)]

    cps = fire(0)
    for c in range(NCHUNK):
        for cp in cps:
            cp.wait()
        cps = fire(c + 1) if c + 1 < NCHUNK else []
        u_r, p_r, n_r, e1_r, e2_r, e3_r, e4_r = banks[c % 2]

        def pair_step(p, _):
            cf = jnp.zeros((LANES,), jnp.float32)
            it = jnp.zeros((LANES,), jnp.float32)
            us = jnp.zeros((LANES,), jnp.float32)
            for d in range(D // LANES):
                ds_ = pl.ds(d * LANES, LANES)
                xu = u_r[p, ds_]
                xi = p_r[p, ds_]
                cf += xu * (xi - n_r[p, ds_])
                dip = xi - e1_r[p, ds_]
                din = xi - e2_r[p, ds_]
                it += dip * dip - din * din
                dup = xu - e3_r[p, ds_]
                dun = xu - e4_r[p, ds_]
                us += dup * dup - dun * dun
            jj = p & (LANES - 1)
            colv = jnp.full((LANES,), jj, jnp.int32)
            plsc.store_scatter(st_cf, [lane, colv], cf)
            plsc.store_scatter(st_it, [lane, colv], it)
            plsc.store_scatter(st_us, [lane, colv], us)

            @pl.when(jj == LANES - 1)
            def _():
                o = pl.ds(c * CH + p - (LANES - 1), LANES)
                for st, out_b in ((st_cf, cf_b), (st_it, it_b), (st_us, us_b)):
                    rows_ = [st[l, :] for l in range(LANES)]
                    while len(rows_) > 1:
                        rows_ = [a + b for a, b in zip(rows_[::2], rows_[1::2])]
                    out_b[o] = rows_[0]

            return 0

        lax.fori_loop(0, CH, pair_step, 0, unroll=2)

    pltpu.sync_copy(cf_b, cf_hbm.at[pl.ds(base, BPW)])
    pltpu.sync_copy(it_b, it_hbm.at[pl.ds(base, BPW)])
    pltpu.sync_copy(us_b, us_hbm.at[pl.ds(base, BPW)])


def _sc_partials(x_rm, idx_cols):
    mesh = plsc.VectorSubcoreMesh(core_axis_name="c", subcore_axis_name="s")
    f32 = jnp.float32
    out = jax.ShapeDtypeStruct((B,), f32)
    rows = pltpu.VMEM((CH, D), f32)
    stage = pltpu.VMEM((LANES, LANES), f32)
    outb = pltpu.VMEM((BPW,), f32)
    return pl.kernel(
        _sc_body,
        out_type=(out, out, out),
        mesh=mesh,
        compiler_params=pltpu.CompilerParams(use_tc_tiling_on_sc=False,
                                             needs_layout_passes=False),
        scratch_types=(
            [pltpu.VMEM((7, NCHUNK, CH), jnp.int32)]
            + [rows] * 14
            + [stage] * 3
            + [outb] * 3
            + [pltpu.SemaphoreType.DMA] * 14
        ),
    )(x_rm, *idx_cols)


def _log_sigmoid(z):
    return jnp.minimum(z, 0.0) - jnp.log1p(jnp.exp(-jnp.abs(z)))


def _final_body(cf_ref, it_ref, us_ref, mi_ref, mu_ref, ewc_ref, out_ref):
    cf_loss = -jnp.sum(_log_sigmoid(cf_ref[...]))
    reg_loss = (-jnp.sum(_log_sigmoid(it_ref[...] * mi_ref[...]))
                - jnp.sum(_log_sigmoid(us_ref[...] * mu_ref[...])))
    out_ref[0, 0] = cf_loss + ENTITY_AWARE_COFF * reg_loss + ewc_ref[0, 0]


def _finalize(cf, it, us, mi, mu, ewc):
    vec = pl.BlockSpec((B,), lambda: (0,))
    return pl.pallas_call(
        _final_body,
        in_specs=[vec, vec, vec, vec, vec,
                  pl.BlockSpec(memory_space=pltpu.SMEM)],
        out_specs=pl.BlockSpec(memory_space=pltpu.SMEM),
        out_shape=jax.ShapeDtypeStruct((1, 1), jnp.float32),
    )(cf, it, us, mi, mu, ewc)


@jax.jit
def kernel(x, pos_neg_pair_t, x_estimated_mean):
    sh = BLK_M.bit_length() - 1
    half = BLK_M // 2
    idx_cols = []
    for j in (0, 1, 2, 3, 4, 6, 7):
        v = pos_neg_pair_t[:, j]
        v = ((v >> sh) << sh) | ((v & (half - 1)) << 1) | ((v >> (sh - 1)) & 1)
        idx_cols.append(v.reshape(NW, NCHUNK, CH))
    mi = pos_neg_pair_t[:, 5].astype(jnp.float32)
    mu = pos_neg_pair_t[:, 8].astype(jnp.float32)

    x_rm, ewc = _prep(x.T, x_estimated_mean.T)
    cf, it, us = _sc_partials(x_rm.reshape(N2, D), idx_cols)
    loss = _finalize(cf, it, us, mi, mu, ewc)
    return loss[0, 0]
